# Initial kernel scaffold; baseline (speedup 1.0000x reference)
import jax
import jax.numpy as jnp
from jax.experimental import pallas as pl


def kernel(query, query_pos, reference_points, feat0, feat1, feat2, lidar2img, W_attn, b_attn, W_out, b_out, W_pe1, b_pe1, W_pe2, b_pe2, img_h, img_w):
    raise NotImplementedError("write your pallas kernel here")


# speedup vs baseline: 1.7105x; 1.7105x over previous
import functools

import jax
import jax.numpy as jnp
from jax import lax
from jax.experimental import pallas as pl
from jax.experimental.pallas import tpu as pltpu
from jax.experimental.pallas import tpu_sc as plsc

_PC_RANGE = (-51.2, -51.2, -5.0, 51.2, 51.2, 3.0)
_EMBED = 256
_NCAMS = 6
_NLEV = 3
_LEVEL_HW = ((58, 100), (29, 50), (15, 25))

_NTEC = 32
_ROWS_PER_Q = _NCAMS * _NLEV * 4


def _build_table(feat0, feat1, feat2):
    tabs = []
    for feat in (feat0, feat1, feat2):
        B, N, C, H, W = feat.shape
        tabs.append(jnp.transpose(feat, (0, 1, 3, 4, 2)).reshape(B * N * H * W, C))
    return jnp.concatenate(tabs, axis=0)


def _build_indices(query, query_pos, reference_points, lidar2img, W_attn, b_attn,
                   img_h, img_w):
    B, Q, _ = query.shape
    attn = jax.nn.sigmoid((query + query_pos) @ W_attn.T + b_attn)
    attn = attn.reshape(B, Q, _NCAMS, _NLEV)

    pc = _PC_RANGE
    rp = jnp.stack([
        reference_points[..., 0] * (pc[3] - pc[0]) + pc[0],
        reference_points[..., 1] * (pc[4] - pc[1]) + pc[1],
        reference_points[..., 2] * (pc[5] - pc[2]) + pc[2],
        jnp.ones_like(reference_points[..., 0])], axis=-1)
    rp_cam = jnp.einsum('bnij,bqj->bnqi', lidar2img, rp)
    eps = 1e-5
    depth_ok = rp_cam[..., 2] > eps
    denom = jnp.maximum(rp_cam[..., 2], eps)
    gx = (rp_cam[..., 0] / denom / img_w - 0.5) * 2.0
    gy = (rp_cam[..., 1] / denom / img_h - 0.5) * 2.0
    mask = depth_ok & (gx > -1.0) & (gx < 1.0) & (gy > -1.0) & (gy < 1.0)
    mask_f = mask.astype(jnp.float32)

    bn = (jnp.arange(B * _NCAMS, dtype=jnp.int32)
          .reshape(B, _NCAMS, 1))
    idx_parts, wt_parts = [], []
    row_base = 0
    for lvl, (H, W) in enumerate(_LEVEL_HW):
        xi = (gx + 1.0) * W / 2.0 - 0.5
        yi = (gy + 1.0) * H / 2.0 - 0.5
        x0 = jnp.floor(xi)
        y0 = jnp.floor(yi)
        attn_l = jnp.transpose(attn[:, :, :, lvl], (0, 2, 1))
        for dx, dy in ((0, 0), (1, 0), (0, 1), (1, 1)):
            xc = x0 + dx
            yc = y0 + dy
            valid = ((xc >= 0) & (xc <= W - 1) & (yc >= 0) & (yc <= H - 1))
            wcorner = (1.0 - jnp.abs(xi - xc)) * (1.0 - jnp.abs(yi - yc))
            xcc = jnp.clip(xc, 0, W - 1).astype(jnp.int32)
            ycc = jnp.clip(yc, 0, H - 1).astype(jnp.int32)
            idx_parts.append(row_base + bn * (H * W) + ycc * W + xcc)
            wt_parts.append(wcorner * valid.astype(jnp.float32) * mask_f * attn_l)
        row_base += B * _NCAMS * H * W
    idx = jnp.stack(idx_parts, axis=0)
    wt = jnp.stack(wt_parts, axis=0)
    idx = jnp.transpose(idx, (1, 3, 2, 0)).reshape(B * Q, _ROWS_PER_Q)
    wt = jnp.transpose(wt, (1, 3, 2, 0)).reshape(B * Q, _ROWS_PER_Q)
    return idx, wt


_WPAD = 80


def _sc_gather_combine(table, idx, wt, qpad, qpt):
    mesh = plsc.VectorSubcoreMesh(core_axis_name="c", subcore_axis_name="s")

    @functools.partial(
        pl.kernel, mesh=mesh,
        out_type=jax.ShapeDtypeStruct((_NTEC, qpt, _EMBED), jnp.float32),
        scratch_types=[
            pltpu.VMEM((qpt, _ROWS_PER_Q), jnp.int32),
            pltpu.VMEM((qpt, _WPAD), jnp.float32),
            pltpu.VMEM((_ROWS_PER_Q, _EMBED), jnp.float32),
            pltpu.VMEM((qpt, _EMBED), jnp.float32),
            pltpu.SemaphoreType.DMA,
        ],
    )
    def k(table_hbm, idx_hbm, wt_hbm, out_hbm, idx_v, wt_v, rows_v, out_v, sem):
        c = lax.axis_index("c")
        s = lax.axis_index("s")
        wid = s * 2 + c
        pltpu.sync_copy(idx_hbm.at[wid], idx_v)
        pltpu.sync_copy(wt_hbm.at[wid], wt_v)

        def fma_block(accs, wvec, row0, njj):
            for jj in range(njj):
                w = wvec[jj]
                accs = tuple(accs[t] + w * rows_v[row0 + jj, pl.ds(t * 16, 16)]
                             for t in range(16))
            return accs

        def body_q(qi, carry):
            pltpu.async_copy(table_hbm.at[idx_v.at[qi]], rows_v, sem).wait()

            def body_jb(jb, accs):
                wvec = wt_v[qi, pl.ds(jb * 16, 16)]
                return fma_block(accs, wvec, jb * 16, 16)

            accs = lax.fori_loop(
                0, 4, body_jb,
                tuple(jnp.zeros((16,), jnp.float32) for _ in range(16)))
            wvec = wt_v[qi, pl.ds(64, 16)]
            accs = fma_block(accs, wvec, 64, 8)
            for t in range(16):
                out_v[qi, pl.ds(t * 16, 16)] = accs[t]
            return carry

        lax.fori_loop(0, qpt, body_q, 0)
        pltpu.sync_copy(out_v, out_hbm.at[wid])

    idx = idx.reshape(_NTEC, qpt, _ROWS_PER_Q)
    wt = wt.reshape(_NTEC, qpt, _WPAD)
    return k(table, idx, wt).reshape(qpad, _EMBED)


def kernel(query, query_pos, reference_points, feat0, feat1, feat2, lidar2img,
           W_attn, b_attn, W_out, b_out, W_pe1, b_pe1, W_pe2, b_pe2, img_h, img_w):
    B, Q, D = query.shape
    qpt = -(-(B * Q) // _NTEC)
    qpad = qpt * _NTEC

    table = _build_table(feat0, feat1, feat2)
    idx, wt = _build_indices(query, query_pos, reference_points, lidar2img,
                             W_attn, b_attn, img_h, img_w)
    pad = qpad - B * Q
    idx = jnp.concatenate([idx, jnp.zeros((pad, _ROWS_PER_Q), jnp.int32)], axis=0)
    wt = jnp.concatenate([wt, jnp.zeros((pad, _ROWS_PER_Q), jnp.float32)], axis=0)
    wt = jnp.pad(wt, ((0, 0), (0, _WPAD - _ROWS_PER_Q)))

    fused = _sc_gather_combine(table, idx, wt, qpad, qpt)[:B * Q]
    fused = fused.reshape(B, Q, _EMBED)

    out = fused @ W_out.T + b_out

    x = jnp.clip(reference_points, 0.0, 1.0)
    x1 = jnp.clip(x, 1e-5, None)
    x2 = jnp.clip(1.0 - x, 1e-5, None)
    inv = jnp.log(x1 / x2)
    pos = jax.nn.relu(inv @ W_pe1.T + b_pe1) @ W_pe2.T + b_pe2
    return out + pos
